# Initial kernel scaffold; baseline (speedup 1.0000x reference)
import jax
import jax.numpy as jnp
from jax.experimental import pallas as pl


def kernel(node_features, edge_attr, W_coord, W_f, b_f, ln_g, ln_b, oln_g, oln_b, W_out, b_out, edge_index):
    raise NotImplementedError("write your pallas kernel here")


# speedup vs baseline: 3.2808x; 3.2808x over previous
import functools

import jax
import jax.numpy as jnp
from jax import lax
from jax.experimental import pallas as pl
from jax.experimental.pallas import tpu as pltpu
from jax.experimental.pallas import tpu_sc as plsc

_NC = 2
_NS = 16
_LANES = 16
_K = 80
_NBLK = 10


def _zero_vec():
    return jnp.zeros((_LANES,), jnp.float32)


def _fill_rows(ref, nrows, ncols, value):
    v = jnp.full((_LANES,), value, jnp.float32)

    def body(r, _):
        for c in range(ncols // _LANES):
            ref[r, pl.ds(c * _LANES, _LANES)] = v
        return 0

    lax.fori_loop(0, nrows, body, 0)


def _fill_flat(ref, n, value):
    v = jnp.full((_LANES,), value, jnp.float32)

    def body(i, _):
        ref[pl.ds(i * _LANES, _LANES)] = v
        return 0

    lax.fori_loop(0, n // _LANES, body, 0)


def _sc_pre(N, E, D, DC,
            coords, ea, src, dst,
            ea_out, deg0_out, deg1_out, s_out,
            acc_ea, acc_deg,
            a_buf, b_buf, ea_buf, s_all, ones_v, src_v, dst_v,
            gsem):
    c = lax.axis_index("c")
    s = lax.axis_index("s")
    ew = E // (_NC * _NS)
    base = (c * _NS + s) * ew
    nstr = N // _NS

    _fill_rows(ea_buf, _K, D, 0.0)
    _fill_flat(s_all, ew, 0.0)
    _fill_flat(ones_v, _K, 1.0)

    @pl.when(s < 10)
    def _():
        for j in range(12):
            pltpu.sync_copy(ea_buf, acc_ea.at[pl.ds(s * 1000 + j * _K, _K)])
        pltpu.sync_copy(ea_buf.at[pl.ds(0, 40)],
                        acc_ea.at[pl.ds(s * 1000 + 960, 40)])

    @pl.when(s == 0)
    def _():
        pltpu.sync_copy(s_all, acc_deg.at[pl.ds(0, ew)])

    plsc.subcore_barrier()

    rows16 = lax.iota(jnp.int32, _LANES)

    def chunk(g, _):
        off = base + g * _K
        pltpu.sync_copy(src.at[pl.ds(off, _K)], src_v)
        pltpu.sync_copy(dst.at[pl.ds(off, _K)], dst_v)
        ca = pltpu.async_copy(coords.at[src_v], a_buf, gsem)
        cb = pltpu.async_copy(coords.at[dst_v], b_buf, gsem)
        ce = pltpu.async_copy(ea.at[pl.ds(off, _K)], ea_buf, gsem)
        ca.wait()
        cb.wait()
        for m in range(_K // _LANES):
            s16 = _zero_vec()
            for j in range(_LANES):
                k = m * _LANES + j
                ra = a_buf[k, pl.ds(0, _LANES)]
                rb = b_buf[k, pl.ds(0, _LANES)]
                prod = ra * rb
                dot8 = ((prod[0] + prod[1]) + (prod[2] + prod[3])
                        + ((prod[4] + prod[5]) + (prod[6] + prod[7])))
                sval = ra[DC] + rb[DC] - 2.0 * dot8
                s16 = jnp.where(rows16 == j, sval, s16)
            s_all[pl.ds(g * _K + m * _LANES, _LANES)] = s16
        ce.wait()
        pltpu.sync_copy(ea_buf, acc_ea.at[dst_v], add=True)
        pltpu.sync_copy(ones_v, acc_deg.at[dst_v], add=True)
        return 0

    lax.fori_loop(0, ew // _K, chunk, 0)
    plsc.subcore_barrier()

    pltpu.sync_copy(s_all, s_out.at[pl.ds(base, ew)])
    @pl.when(s < 10)
    def _():
        pltpu.sync_copy(acc_ea.at[pl.ds(s * 1000, 1000)],
                        ea_out.at[c, pl.ds(s * 1000, 1000)])

    @pl.when((s == 0) & (c == 0))
    def _():
        pltpu.sync_copy(acc_deg, deg0_out)

    @pl.when((s == 0) & (c == 1))
    def _():
        pltpu.sync_copy(acc_deg, deg1_out)


def _sc_edge(N, E, D, with_msg,
             *refs):
    if with_msg:
        (xf, src, dst, t, ea,
         parts_out, msg_out,
         acc, rows, ea_buf, t_all, src_v, dst_v, gsem) = refs
    else:
        (xf, src, dst, t,
         parts_out,
         acc, rows, t_all, src_v, dst_v, gsem) = refs
        ea = ea_buf = msg_out = None
    c = lax.axis_index("c")
    s = lax.axis_index("s")
    ew = E // (_NC * _NS)
    base = (c * _NS + s) * ew
    nstr = N // _NS

    _fill_rows(rows, _K, D, 0.0)

    @pl.when(s < 10)
    def _():
        for j in range(12):
            pltpu.sync_copy(rows, acc.at[pl.ds(s * 1000 + j * _K, _K)])
        pltpu.sync_copy(rows.at[pl.ds(0, 40)],
                        acc.at[pl.ds(s * 1000 + 960, 40)])

    pltpu.sync_copy(t.at[pl.ds(base, ew)], t_all)
    plsc.subcore_barrier()

    def chunk(g, _):
        off = base + g * _K
        pltpu.sync_copy(src.at[pl.ds(off, _K)], src_v)
        pltpu.sync_copy(dst.at[pl.ds(off, _K)], dst_v)
        cg = pltpu.async_copy(xf.at[src_v], rows, gsem)
        if with_msg:
            ce = pltpu.async_copy(ea.at[pl.ds(off, _K)], ea_buf, gsem)
        cg.wait()
        if with_msg:
            ce.wait()

        for m in range(_K // _LANES):
            tv16 = t_all[pl.ds(g * _K + m * _LANES, _LANES)]
            for j in range(_LANES):
                k = m * _LANES + j
                tv = jnp.full((_LANES,), tv16[j], jnp.float32)
                for d in range(D // _LANES):
                    sl = pl.ds(d * _LANES, _LANES)
                    if with_msg:
                        rows[k, sl] = rows[k, sl] * tv + ea_buf[k, sl]
                    else:
                        rows[k, sl] = rows[k, sl] * tv
        if with_msg:
            pltpu.sync_copy(rows, msg_out.at[pl.ds(off, _K)])
        pltpu.sync_copy(rows, acc.at[dst_v], add=True)
        return 0

    lax.fori_loop(0, ew // _K, chunk, 0)
    plsc.subcore_barrier()

    @pl.when(s < 10)
    def _():
        pltpu.sync_copy(acc.at[pl.ds(s * 1000, 1000)],
                        parts_out.at[c, pl.ds(s * 1000, 1000)])


def _tc_pre_body(DC, x_ref, wc_ref, w0_ref, b0_ref, caug_ref, xf0_ref):
    x = x_ref[...]
    caug = jnp.dot(x, wc_ref[...], preferred_element_type=jnp.float32)
    q = jnp.sum(caug * caug, axis=-1, keepdims=True)
    lane = lax.broadcasted_iota(jnp.int32, caug.shape, 1)
    caug_ref[...] = jnp.where(lane == DC, q, caug)
    xf0_ref[...] = (jnp.dot(x, w0_ref[...], preferred_element_type=jnp.float32)
                    + b0_ref[...])


def _tc_t_body(s_ref, t_ref):
    t_ref[...] = jnp.exp(-jnp.sqrt(jnp.maximum(s_ref[...], 0.0)))


def _ln(h, g, b, eps=1e-5):
    m = jnp.mean(h, axis=-1, keepdims=True)
    v = jnp.mean((h - m) ** 2, axis=-1, keepdims=True)
    return (h - m) / jnp.sqrt(v + eps) * g + b


def _tc_post_body(with_ea,
                  x_ref, p_ref, ea_ref, d0_ref, d1_ref, g_ref, b_ref,
                  w_ref, bn_ref, xn_ref, xfn_ref):
    upd = p_ref[0] + p_ref[1]
    if with_ea:
        upd = upd + ea_ref[0] + ea_ref[1]
    deg = jnp.maximum(d0_ref[...] + d1_ref[...], 1.0)
    xn = _ln(x_ref[...] + upd / deg, g_ref[...], b_ref[...])
    xn_ref[...] = xn
    xfn_ref[...] = (jnp.dot(xn, w_ref[...], preferred_element_type=jnp.float32)
                    + bn_ref[...])


def _tc_final_body(x_ref, p_ref, d0_ref, d1_ref, g_ref, b_ref,
                   og_ref, ob_ref, wo_ref, bo_ref, out_ref):
    upd = p_ref[0] + p_ref[1]
    deg = jnp.maximum(d0_ref[...] + d1_ref[...], 1.0)
    xn = _ln(x_ref[...] + upd / deg, g_ref[...], b_ref[...])
    xo = _ln(xn, og_ref[...], ob_ref[...])
    out_ref[...] = (jnp.dot(xo, wo_ref[...], preferred_element_type=jnp.float32)
                    + bo_ref[...])


def kernel(node_features, edge_attr, W_coord, W_f, b_f, ln_g, ln_b,
           oln_g, oln_b, W_out, b_out, edge_index):
    B, N, D = node_features.shape
    E = edge_index.shape[1]
    DC = W_coord.shape[1]
    L = W_f.shape[0]
    assert B == 1 and D == 128 and N == 10000 and E == 320000 and L == 3

    x0 = node_features.reshape(N, D)
    src = edge_index[0]
    dst = edge_index[1]
    bs = N // _NBLK

    f32 = jnp.float32
    mesh = plsc.VectorSubcoreMesh(core_axis_name="c", subcore_axis_name="s")
    ew = E // (_NC * _NS)

    wc_pad = jnp.zeros((D, D), f32).at[:, :DC].set(W_coord)
    caug, xf0 = pl.pallas_call(
        functools.partial(_tc_pre_body, DC),
        grid=(_NBLK,),
        in_specs=[
            pl.BlockSpec((bs, D), lambda i: (i, 0)),
            pl.BlockSpec((D, D), lambda i: (0, 0)),
            pl.BlockSpec((D, D), lambda i: (0, 0)),
            pl.BlockSpec((1, D), lambda i: (0, 0)),
        ],
        out_specs=[
            pl.BlockSpec((bs, D), lambda i: (i, 0)),
            pl.BlockSpec((bs, D), lambda i: (i, 0)),
        ],
        out_shape=[
            jax.ShapeDtypeStruct((N, D), f32),
            jax.ShapeDtypeStruct((N, D), f32),
        ],
    )(x0, wc_pad, W_f[0], b_f[0][None, :])

    ea_parts, deg0, deg1, s_flat = pl.kernel(
        functools.partial(_sc_pre, N, E, D, DC),
        out_type=[
            jax.ShapeDtypeStruct((_NC, N, D), f32),
            jax.ShapeDtypeStruct((N,), f32),
            jax.ShapeDtypeStruct((N,), f32),
            jax.ShapeDtypeStruct((E,), f32),
        ],
        mesh=mesh,
        scratch_types=[
            pltpu.VMEM_SHARED((N, D), f32),
            pltpu.VMEM_SHARED((N,), f32),
            pltpu.VMEM((_K, D), f32),
            pltpu.VMEM((_K, D), f32),
            pltpu.VMEM((_K, D), f32),
            pltpu.VMEM((ew,), f32),
            pltpu.VMEM((_K,), f32),
            pltpu.VMEM((_K,), jnp.int32),
            pltpu.VMEM((_K,), jnp.int32),
            pltpu.SemaphoreType.DMA,
        ],
    )(caug, edge_attr, src, dst)

    t2d = pl.pallas_call(
        _tc_t_body,
        grid=(1,),
        in_specs=[pl.BlockSpec((E // D, D), lambda i: (0, 0))],
        out_specs=pl.BlockSpec((E // D, D), lambda i: (0, 0)),
        out_shape=jax.ShapeDtypeStruct((E // D, D), f32),
    )(s_flat.reshape(E // D, D))
    t = t2d.reshape(E)

    d0 = deg0.reshape(N, 1)
    d1 = deg1.reshape(N, 1)

    def sc_edge_layer(xf):
        return pl.kernel(
            functools.partial(_sc_edge, N, E, D, False),
            out_type=jax.ShapeDtypeStruct((_NC, N, D), f32),
            mesh=mesh,
            scratch_types=[
                pltpu.VMEM_SHARED((N, D), f32),
                pltpu.VMEM((_K, D), f32),
                pltpu.VMEM((ew,), f32),
                pltpu.VMEM((_K,), jnp.int32),
                pltpu.VMEM((_K,), jnp.int32),
                pltpu.SemaphoreType.DMA,
            ],
        )(xf, src, dst, t)

    def tc_post(x, parts, wl, bl, gl, blb):
        return pl.pallas_call(
            functools.partial(_tc_post_body, True),
            grid=(_NBLK,),
            in_specs=[
                pl.BlockSpec((bs, D), lambda i: (i, 0)),
                pl.BlockSpec((_NC, bs, D), lambda i: (0, i, 0)),
                pl.BlockSpec((_NC, bs, D), lambda i: (0, i, 0)),
                pl.BlockSpec((bs, 1), lambda i: (i, 0)),
                pl.BlockSpec((bs, 1), lambda i: (i, 0)),
                pl.BlockSpec((1, D), lambda i: (0, 0)),
                pl.BlockSpec((1, D), lambda i: (0, 0)),
                pl.BlockSpec((D, D), lambda i: (0, 0)),
                pl.BlockSpec((1, D), lambda i: (0, 0)),
            ],
            out_specs=[
                pl.BlockSpec((bs, D), lambda i: (i, 0)),
                pl.BlockSpec((bs, D), lambda i: (i, 0)),
            ],
            out_shape=[
                jax.ShapeDtypeStruct((N, D), f32),
                jax.ShapeDtypeStruct((N, D), f32),
            ],
        )(x, parts, ea_parts, d0, d1, gl[None, :], blb[None, :],
          wl, bl[None, :])

    parts0 = sc_edge_layer(xf0)
    x1, xf1 = tc_post(x0, parts0, W_f[1], b_f[1], ln_g[0], ln_b[0])
    parts1 = sc_edge_layer(xf1)
    x2, xf2 = tc_post(x1, parts1, W_f[2], b_f[2], ln_g[1], ln_b[1])

    parts2, msg = pl.kernel(
        functools.partial(_sc_edge, N, E, D, True),
        out_type=[
            jax.ShapeDtypeStruct((_NC, N, D), f32),
            jax.ShapeDtypeStruct((E, D), f32),
        ],
        mesh=mesh,
        scratch_types=[
            pltpu.VMEM_SHARED((N, D), f32),
            pltpu.VMEM((_K, D), f32),
            pltpu.VMEM((_K, D), f32),
            pltpu.VMEM((ew,), f32),
            pltpu.VMEM((_K,), jnp.int32),
            pltpu.VMEM((_K,), jnp.int32),
            pltpu.SemaphoreType.DMA,
        ],
    )(xf2, src, dst, t, edge_attr)

    node_out = pl.pallas_call(
        _tc_final_body,
        grid=(_NBLK,),
        in_specs=[
            pl.BlockSpec((bs, D), lambda i: (i, 0)),
            pl.BlockSpec((_NC, bs, D), lambda i: (0, i, 0)),
            pl.BlockSpec((bs, 1), lambda i: (i, 0)),
            pl.BlockSpec((bs, 1), lambda i: (i, 0)),
            pl.BlockSpec((1, D), lambda i: (0, 0)),
            pl.BlockSpec((1, D), lambda i: (0, 0)),
            pl.BlockSpec((1, D), lambda i: (0, 0)),
            pl.BlockSpec((1, D), lambda i: (0, 0)),
            pl.BlockSpec((D, D), lambda i: (0, 0)),
            pl.BlockSpec((1, D), lambda i: (0, 0)),
        ],
        out_specs=pl.BlockSpec((bs, D), lambda i: (i, 0)),
        out_shape=jax.ShapeDtypeStruct((N, D), f32),
    )(x2, parts2, d0, d1, ln_g[2][None, :], ln_b[2][None, :],
      oln_g[None, :], oln_b[None, :], W_out, b_out[None, :])

    return node_out.reshape(1, N, D), msg.reshape(1, E, D)
